# baseline (speedup 1.0000x reference)
import math

import jax
import jax.numpy as jnp
from jax import lax
from jax.experimental import pallas as pl
from jax.experimental.pallas import tpu as pltpu
from jax.experimental.pallas import tpu_sc as plsc
import numpy as np

_VOCAB = 1000000
_D = 128
_MAX_LEN = 200
_NUM_LAYERS = 6
_B = 4096
_L = 200

_NC = 2
_NS = 16
_NW = _NC * _NS

_ROWS = _B * _L
_RPW = _ROWS // _NW
_CHUNK = 128
_NCHUNK = _RPW // _CHUNK
_SCALE = math.sqrt(float(_D))
_LANES = 16
_VPR = _D // _LANES


def _sincos_table(max_len, d_model):
    pe = np.zeros((max_len, d_model), dtype=np.float32)
    pos = np.arange(max_len, dtype=np.float64)[:, None]
    i = np.arange(0, d_model, 2, dtype=np.float64)
    pe[:, 0::2] = np.sin(pos / np.power(10000.0, 2.0 * i / d_model)).astype(np.float32)
    pe[:, 1::2] = np.cos(pos / np.power(10000.0, 2.0 * (i + 1.0) / d_model)).astype(np.float32)
    return pe


_PE = _sincos_table(_MAX_LEN, _D)
_TE = _sincos_table(_NUM_LAYERS, _D)


def _body(table_hbm, idx_hbm, pe_hbm, out_hbm, idx_v, buf, pe_v, sem):
    wid = lax.axis_index("s") * _NC + lax.axis_index("c")
    pltpu.sync_copy(pe_hbm, pe_v)

    def chunk_body(c, carry):
        base = wid * _RPW + c * _CHUNK
        pltpu.sync_copy(idx_hbm.at[pl.ds(base, _CHUNK)], idx_v)
        pltpu.async_copy(table_hbm.at[idx_v], buf, sem).wait()
        pos0 = lax.rem(c * _CHUNK, _MAX_LEN)

        def row_body(i, carry2):
            l = pos0 + i
            l = jnp.where(l >= _MAX_LEN, l - _MAX_LEN, l)
            for j in range(_VPR):
                sl = pl.ds(j * _LANES, _LANES)
                buf[i, sl] = buf[i, sl] * _SCALE + pe_v[l, sl]
            return carry2

        lax.fori_loop(0, _CHUNK, row_body, 0)
        pltpu.sync_copy(buf, out_hbm.at[pl.ds(base, _CHUNK), :])
        return carry

    lax.fori_loop(0, _NCHUNK, chunk_body, 0)


def kernel(embedding, layer_idx, embed_weight):
    pe = jnp.asarray(_PE)
    te_row = jnp.take(jnp.asarray(_TE), layer_idx, axis=0)
    pe_c = pe + te_row[None, :]

    idx_flat = embedding.reshape(_ROWS).astype(jnp.int32)

    mesh = plsc.VectorSubcoreMesh(core_axis_name="c", subcore_axis_name="s")
    out = pl.kernel(
        _body,
        out_type=jax.ShapeDtypeStruct((_ROWS, _D), jnp.float32),
        mesh=mesh,
        scratch_types=[
            pltpu.VMEM((_CHUNK,), jnp.int32),
            pltpu.VMEM((_CHUNK, _D), jnp.float32),
            pltpu.VMEM((_MAX_LEN, _D), jnp.float32),
            pltpu.SemaphoreType.DMA,
        ],
    )(embed_weight, idx_flat, pe_c)
    return out.reshape(_B, _L, _D)


# speedup vs baseline: 1.5200x; 1.5200x over previous
import math

import jax
import jax.numpy as jnp
from jax import lax
from jax.experimental import pallas as pl
from jax.experimental.pallas import tpu as pltpu
from jax.experimental.pallas import tpu_sc as plsc
import numpy as np

_VOCAB = 1000000
_D = 128
_MAX_LEN = 200
_NUM_LAYERS = 6
_B = 4096
_L = 200

_NC = 2
_NS = 16
_NW = _NC * _NS

_ROWS = _B * _L
_RPW = _ROWS // _NW
_CHUNK = 128
_NCHUNK = _RPW // _CHUNK
_SCALE = math.sqrt(float(_D))
_LANES = 16
_VPR = _D // _LANES


def _sincos_table(max_len, d_model):
    pe = np.zeros((max_len, d_model), dtype=np.float32)
    pos = np.arange(max_len, dtype=np.float64)[:, None]
    i = np.arange(0, d_model, 2, dtype=np.float64)
    pe[:, 0::2] = np.sin(pos / np.power(10000.0, 2.0 * i / d_model)).astype(np.float32)
    pe[:, 1::2] = np.cos(pos / np.power(10000.0, 2.0 * (i + 1.0) / d_model)).astype(np.float32)
    return pe


_PE = _sincos_table(_MAX_LEN, _D)
_TE = _sincos_table(_NUM_LAYERS, _D)


def _body(table_hbm, idx_hbm, pe_hbm, out_hbm,
          idx0, idx1, g0, g1, o0, o1, pe_v,
          gs0, gs1, ss0, ss1, is0, is1):
    idxv, gbuf, obuf = [idx0, idx1], [g0, g1], [o0, o1]
    gsem, ssem, isem = [gs0, gs1], [ss0, ss1], [is0, is1]

    wid = lax.axis_index("s") * _NC + lax.axis_index("c")
    base0 = wid * _RPW
    pltpu.sync_copy(pe_hbm, pe_v)

    for b in range(2):
        pltpu.sync_copy(idx_hbm.at[pl.ds(base0 + b * _CHUNK, _CHUNK)], idxv[b])
        pltpu.make_async_copy(table_hbm.at[idxv[b]], gbuf[b], gsem[b]).start()

    def outer(k, carry):
        g = k * 2
        for b in range(2):
            c = g + b
            base = base0 + c * _CHUNK
            pltpu.make_async_copy(table_hbm.at[idxv[b]], gbuf[b], gsem[b]).wait()

            @pl.when(c + 2 < _NCHUNK)
            def _():
                pltpu.make_async_copy(
                    idx_hbm.at[pl.ds(base + 2 * _CHUNK, _CHUNK)], idxv[b],
                    isem[b]).start()

            @pl.when(c >= 2)
            def _():
                pltpu.make_async_copy(
                    obuf[b], out_hbm.at[pl.ds(base - 2 * _CHUNK, _CHUNK), :],
                    ssem[b]).wait()

            pos0 = lax.rem(c * _CHUNK, _MAX_LEN)

            def row_body(i, l):
                for j in range(_VPR):
                    sl = pl.ds(j * _LANES, _LANES)
                    obuf[b][i, sl] = gbuf[b][i, sl] * _SCALE + pe_v[l, sl]
                l = l + 1
                return jnp.where(l >= _MAX_LEN, l - _MAX_LEN, l)

            lax.fori_loop(0, _CHUNK, row_body, pos0)

            pltpu.make_async_copy(
                obuf[b], out_hbm.at[pl.ds(base, _CHUNK), :], ssem[b]).start()

            @pl.when(c + 2 < _NCHUNK)
            def _():
                pltpu.make_async_copy(
                    idx_hbm.at[pl.ds(base + 2 * _CHUNK, _CHUNK)], idxv[b],
                    isem[b]).wait()
                pltpu.make_async_copy(table_hbm.at[idxv[b]], gbuf[b],
                                      gsem[b]).start()
        return carry

    lax.fori_loop(0, _NCHUNK // 2, outer, 0)

    for b in range(2):
        base = base0 + (_NCHUNK - 2 + b) * _CHUNK
        pltpu.make_async_copy(
            obuf[b], out_hbm.at[pl.ds(base, _CHUNK), :], ssem[b]).wait()


def kernel(embedding, layer_idx, embed_weight):
    pe = jnp.asarray(_PE)
    te_row = jnp.take(jnp.asarray(_TE), layer_idx, axis=0)
    pe_c = pe + te_row[None, :]

    idx_flat = embedding.reshape(_ROWS).astype(jnp.int32)

    mesh = plsc.VectorSubcoreMesh(core_axis_name="c", subcore_axis_name="s")
    out = pl.kernel(
        _body,
        out_type=jax.ShapeDtypeStruct((_ROWS, _D), jnp.float32),
        mesh=mesh,
        scratch_types=[
            pltpu.VMEM((_CHUNK,), jnp.int32),
            pltpu.VMEM((_CHUNK,), jnp.int32),
            pltpu.VMEM((_CHUNK, _D), jnp.float32),
            pltpu.VMEM((_CHUNK, _D), jnp.float32),
            pltpu.VMEM((_CHUNK, _D), jnp.float32),
            pltpu.VMEM((_CHUNK, _D), jnp.float32),
            pltpu.VMEM((_MAX_LEN, _D), jnp.float32),
            pltpu.SemaphoreType.DMA,
            pltpu.SemaphoreType.DMA,
            pltpu.SemaphoreType.DMA,
            pltpu.SemaphoreType.DMA,
            pltpu.SemaphoreType.DMA,
            pltpu.SemaphoreType.DMA,
        ],
    )(embed_weight, idx_flat, pe_c)
    return out.reshape(_B, _L, _D)


# speedup vs baseline: 4.5696x; 3.0063x over previous
import math

import jax
import jax.numpy as jnp
from jax import lax
from jax.experimental import pallas as pl
from jax.experimental.pallas import tpu as pltpu
from jax.experimental.pallas import tpu_sc as plsc
import numpy as np

_VOCAB = 1000000
_D = 128
_MAX_LEN = 200
_NUM_LAYERS = 6
_B = 4096
_L = 200

_NC = 2
_NS = 16
_NW = _NC * _NS

_CHUNK = _B // _NW
_SCALE = math.sqrt(float(_D))
_LANES = 16
_VPR = _D // _LANES
_UNROLL = 8


def _sincos_table(max_len, d_model):
    pe = np.zeros((max_len, d_model), dtype=np.float32)
    pos = np.arange(max_len, dtype=np.float64)[:, None]
    i = np.arange(0, d_model, 2, dtype=np.float64)
    pe[:, 0::2] = np.sin(pos / np.power(10000.0, 2.0 * i / d_model)).astype(np.float32)
    pe[:, 1::2] = np.cos(pos / np.power(10000.0, 2.0 * (i + 1.0) / d_model)).astype(np.float32)
    return pe


_PE = _sincos_table(_MAX_LEN, _D)
_TE = _sincos_table(_NUM_LAYERS, _D)


def _body(table_hbm, idx_hbm, pe_hbm, out_hbm,
          idx0, idx1, g0, g1, o0, o1, pe_v,
          gs0, gs1, ss0, ss1, is0, is1):
    idxv, gbuf, obuf = [idx0, idx1], [g0, g1], [o0, o1]
    gsem, ssem, isem = [gs0, gs1], [ss0, ss1], [is0, is1]

    wid = lax.axis_index("s") * _NC + lax.axis_index("c")
    b0 = wid * _CHUNK
    pltpu.sync_copy(pe_hbm, pe_v)

    for b in range(2):
        pltpu.sync_copy(idx_hbm.at[pl.ds(b * _B + b0, _CHUNK)], idxv[b])
        pltpu.make_async_copy(table_hbm.at[idxv[b]], gbuf[b], gsem[b]).start()

    def outer(k, carry):
        for b in range(2):
            c = k * 2 + b
            pltpu.make_async_copy(table_hbm.at[idxv[b]], gbuf[b], gsem[b]).wait()

            @pl.when(c + 2 < _L)
            def _():
                pltpu.make_async_copy(
                    idx_hbm.at[pl.ds((c + 2) * _B + b0, _CHUNK)], idxv[b],
                    isem[b]).start()

            @pl.when(c >= 2)
            def _():
                pltpu.make_async_copy(
                    obuf[b],
                    out_hbm.at[pl.ds(b0, _CHUNK), pl.ds(c - 2, 1), :],
                    ssem[b]).wait()

            vp = [pe_v[c, pl.ds(j * _LANES, _LANES)] for j in range(_VPR)]

            def blk(t, carry2):
                for r in range(_UNROLL):
                    i = t * _UNROLL + r
                    for j in range(_VPR):
                        sl = pl.ds(j * _LANES, _LANES)
                        obuf[b][i, 0, sl] = gbuf[b][i, sl] * _SCALE + vp[j]
                return carry2

            lax.fori_loop(0, _CHUNK // _UNROLL, blk, 0)

            pltpu.make_async_copy(
                obuf[b], out_hbm.at[pl.ds(b0, _CHUNK), pl.ds(c, 1), :],
                ssem[b]).start()

            @pl.when(c + 2 < _L)
            def _():
                pltpu.make_async_copy(
                    idx_hbm.at[pl.ds((c + 2) * _B + b0, _CHUNK)], idxv[b],
                    isem[b]).wait()
                pltpu.make_async_copy(table_hbm.at[idxv[b]], gbuf[b],
                                      gsem[b]).start()
        return carry

    lax.fori_loop(0, _L // 2, outer, 0)

    for b in range(2):
        pltpu.make_async_copy(
            obuf[b], out_hbm.at[pl.ds(b0, _CHUNK), pl.ds(_L - 2 + b, 1), :],
            ssem[b]).wait()


def kernel(embedding, layer_idx, embed_weight):
    pe = jnp.asarray(_PE)
    te_row = jnp.take(jnp.asarray(_TE), layer_idx, axis=0)
    pe_c = pe + te_row[None, :]

    idx_t = embedding.astype(jnp.int32).T.reshape(_L * _B)

    mesh = plsc.VectorSubcoreMesh(core_axis_name="c", subcore_axis_name="s")
    out = pl.kernel(
        _body,
        out_type=jax.ShapeDtypeStruct((_B, _L, _D), jnp.float32),
        mesh=mesh,
        scratch_types=[
            pltpu.VMEM((_CHUNK,), jnp.int32),
            pltpu.VMEM((_CHUNK,), jnp.int32),
            pltpu.VMEM((_CHUNK, _D), jnp.float32),
            pltpu.VMEM((_CHUNK, _D), jnp.float32),
            pltpu.VMEM((_CHUNK, 1, _D), jnp.float32),
            pltpu.VMEM((_CHUNK, 1, _D), jnp.float32),
            pltpu.VMEM((_MAX_LEN, _D), jnp.float32),
            pltpu.SemaphoreType.DMA,
            pltpu.SemaphoreType.DMA,
            pltpu.SemaphoreType.DMA,
            pltpu.SemaphoreType.DMA,
            pltpu.SemaphoreType.DMA,
            pltpu.SemaphoreType.DMA,
        ],
    )(embed_weight, idx_t, pe_c)
    return out


# speedup vs baseline: 4.6251x; 1.0122x over previous
import math

import jax
import jax.numpy as jnp
from jax import lax
from jax.experimental import pallas as pl
from jax.experimental.pallas import tpu as pltpu
from jax.experimental.pallas import tpu_sc as plsc
import numpy as np

_VOCAB = 1000000
_D = 128
_MAX_LEN = 200
_NUM_LAYERS = 6
_B = 4096
_L = 200

_NC = 2
_NS = 16
_NW = _NC * _NS

_CHUNK = _B // _NW
_SCALE = math.sqrt(float(_D))
_LANES = 16
_VPR = _D // _LANES
_UNROLL = 8


def _sincos_table(max_len, d_model):
    pe = np.zeros((max_len, d_model), dtype=np.float32)
    pos = np.arange(max_len, dtype=np.float64)[:, None]
    i = np.arange(0, d_model, 2, dtype=np.float64)
    pe[:, 0::2] = np.sin(pos / np.power(10000.0, 2.0 * i / d_model)).astype(np.float32)
    pe[:, 1::2] = np.cos(pos / np.power(10000.0, 2.0 * (i + 1.0) / d_model)).astype(np.float32)
    return pe


_PE = _sincos_table(_MAX_LEN, _D)
_TE = _sincos_table(_NUM_LAYERS, _D)


_NG = 3
_NO = 2
_STEP = 6
_MAIN = (_L // _STEP) * _STEP


def _body(table_hbm, idx_hbm, pe_hbm, out_hbm,
          idx0, idx1, idx2, g0, g1, g2, o0, o1, pe_v,
          gs0, gs1, gs2, ss0, ss1, is0, is1, is2):
    idxv, gbuf, obuf = [idx0, idx1, idx2], [g0, g1, g2], [o0, o1]
    gsem, ssem, isem = [gs0, gs1, gs2], [ss0, ss1], [is0, is1, is2]

    wid = lax.axis_index("s") * _NC + lax.axis_index("c")
    b0 = wid * _CHUNK
    pltpu.sync_copy(pe_hbm, pe_v)

    for g in range(_NG):
        pltpu.sync_copy(idx_hbm.at[pl.ds(g * _B + b0, _CHUNK)], idxv[g])
        pltpu.make_async_copy(table_hbm.at[idxv[g]], gbuf[g], gsem[g]).start()

    def chunk(c, g, o, refill):
        pltpu.make_async_copy(table_hbm.at[idxv[g]], gbuf[g], gsem[g]).wait()

        if refill:
            @pl.when(c + _NG < _L)
            def _():
                pltpu.make_async_copy(
                    idx_hbm.at[pl.ds((c + _NG) * _B + b0, _CHUNK)], idxv[g],
                    isem[g]).start()

        @pl.when(c >= _NO)
        def _():
            pltpu.make_async_copy(
                obuf[o],
                out_hbm.at[pl.ds(b0, _CHUNK), pl.ds(c - _NO, 1), :],
                ssem[o]).wait()

        vp = [pe_v[c, pl.ds(j * _LANES, _LANES)] for j in range(_VPR)]

        def blk(t, carry2):
            for r in range(_UNROLL):
                i = t * _UNROLL + r
                for j in range(_VPR):
                    sl = pl.ds(j * _LANES, _LANES)
                    obuf[o][i, 0, sl] = gbuf[g][i, sl] * _SCALE + vp[j]
            return carry2

        lax.fori_loop(0, _CHUNK // _UNROLL, blk, 0)

        pltpu.make_async_copy(
            obuf[o], out_hbm.at[pl.ds(b0, _CHUNK), pl.ds(c, 1), :],
            ssem[o]).start()

        if refill:
            @pl.when(c + _NG < _L)
            def _():
                pltpu.make_async_copy(
                    idx_hbm.at[pl.ds((c + _NG) * _B + b0, _CHUNK)], idxv[g],
                    isem[g]).wait()
                pltpu.make_async_copy(table_hbm.at[idxv[g]], gbuf[g],
                                      gsem[g]).start()

    def outer(k, carry):
        for u in range(_STEP):
            chunk(k * _STEP + u, u % _NG, u % _NO, refill=True)
        return carry

    lax.fori_loop(0, _MAIN // _STEP, outer, 0)

    for c in range(_MAIN, _L):
        chunk(c, c % _NG, c % _NO, refill=False)

    for c in range(_L - _NO, _L):
        pltpu.make_async_copy(
            obuf[c % _NO], out_hbm.at[pl.ds(b0, _CHUNK), pl.ds(c, 1), :],
            ssem[c % _NO]).wait()


def kernel(embedding, layer_idx, embed_weight):
    pe = jnp.asarray(_PE)
    te_row = jnp.take(jnp.asarray(_TE), layer_idx, axis=0)
    pe_c = pe + te_row[None, :]

    idx_t = embedding.astype(jnp.int32).T.reshape(_L * _B)

    mesh = plsc.VectorSubcoreMesh(core_axis_name="c", subcore_axis_name="s")
    out = pl.kernel(
        _body,
        out_type=jax.ShapeDtypeStruct((_B, _L, _D), jnp.float32),
        mesh=mesh,
        scratch_types=(
            [pltpu.VMEM((_CHUNK,), jnp.int32)] * _NG
            + [pltpu.VMEM((_CHUNK, _D), jnp.float32)] * _NG
            + [pltpu.VMEM((_CHUNK, 1, _D), jnp.float32)] * _NO
            + [pltpu.VMEM((_MAX_LEN, _D), jnp.float32)]
            + [pltpu.SemaphoreType.DMA] * (_NG + _NO + _NG)
        ),
    )(embed_weight, idx_t, pe_c)
    return out
